# Initial kernel scaffold; baseline (speedup 1.0000x reference)
import jax
import jax.numpy as jnp
from jax.experimental import pallas as pl


def kernel(feats, edge_index, W1, b1, W2, b2):
    raise NotImplementedError("write your pallas kernel here")


# speedup vs baseline: 5.8496x; 5.8496x over previous
import functools

import jax
import jax.numpy as jnp
from jax import lax
from jax.experimental import pallas as pl
from jax.experimental.pallas import tpu as pltpu
from jax.experimental.pallas import tpu_sc as plsc

N = 10000
N_PAD = 10240
E = 160000
D_IN = 256
D_HID = 512
D_OUT = 256

NUM_CORES = 2
NUM_SUBCORES = 16
EPW = E // NUM_SUBCORES
RPW = N_PAD // NUM_SUBCORES
CH = 80
NCHUNK = EPW // CH
DEGW = 128


def _sc_mesh():
    return plsc.VectorSubcoreMesh(core_axis_name="c", subcore_axis_name="s")


def _sc_degrees(src, dst, zeros_rows, ones_rows):
    @functools.partial(
        pl.kernel,
        out_type=(jax.ShapeDtypeStruct((N_PAD, DEGW), jnp.float32),
                  jax.ShapeDtypeStruct((N_PAD, DEGW), jnp.float32)),
        mesh=_sc_mesh(),
        scratch_types=[
            pltpu.VMEM_SHARED((N_PAD, DEGW), jnp.float32),
            pltpu.VMEM((CH,), jnp.int32),
            pltpu.VMEM((CH, DEGW), jnp.float32),
        ],
    )
    def deg_kernel(src_hbm, dst_hbm, zeros_hbm, ones_hbm,
                   dsrc_hbm, ddst_hbm, acc, idx, ones_v):
        cid = lax.axis_index("c")
        sid = lax.axis_index("s")
        pltpu.sync_copy(ones_hbm, ones_v)
        pltpu.sync_copy(zeros_hbm.at[pl.ds(sid * RPW, RPW)],
                        acc.at[pl.ds(sid * RPW, RPW)])
        plsc.subcore_barrier()

        def run(idx_hbm, out_hbm):
            def body(j, _):
                base = sid * EPW + j * CH
                pltpu.sync_copy(idx_hbm.at[pl.ds(pl.multiple_of(base, 8), CH)], idx)
                pltpu.sync_copy(ones_v, acc.at[idx], add=True)
                return 0
            lax.fori_loop(0, NCHUNK, body, 0)
            plsc.subcore_barrier()
            pltpu.sync_copy(acc.at[pl.ds(sid * RPW, RPW)],
                            out_hbm.at[pl.ds(sid * RPW, RPW)])

        @pl.when(cid == 0)
        def _():
            run(src_hbm, dsrc_hbm)

        @pl.when(cid == 1)
        def _():
            run(dst_hbm, ddst_hbm)

    return deg_kernel(src, dst, zeros_rows, ones_rows)


def _sc_aggregate(x_lo, x_hi, src, dst):
    @functools.partial(
        pl.kernel,
        out_type=(jax.ShapeDtypeStruct((N_PAD, 128), jnp.float32),
                  jax.ShapeDtypeStruct((N_PAD, 128), jnp.float32)),
        mesh=_sc_mesh(),
        scratch_types=[
            pltpu.VMEM_SHARED((N_PAD, 128), jnp.float32),
            pltpu.VMEM((CH,), jnp.int32),
            pltpu.VMEM((CH,), jnp.int32),
            pltpu.VMEM((CH, 128), jnp.float32),
            pltpu.SemaphoreType.DMA,
        ],
    )
    def agg_kernel(xlo_hbm, xhi_hbm, src_hbm, dst_hbm,
                   olo_hbm, ohi_hbm, acc, sidx, didx, rows, sem):
        cid = lax.axis_index("c")
        sid = lax.axis_index("s")

        def run(x_hbm, o_hbm):
            pltpu.sync_copy(x_hbm.at[pl.ds(sid * RPW, RPW)],
                            acc.at[pl.ds(sid * RPW, RPW)])
            plsc.subcore_barrier()

            def body(j, _):
                base = sid * EPW + j * CH
                pltpu.sync_copy(src_hbm.at[pl.ds(pl.multiple_of(base, 8), CH)], sidx)
                pltpu.sync_copy(dst_hbm.at[pl.ds(pl.multiple_of(base, 8), CH)], didx)
                pltpu.async_copy(x_hbm.at[sidx], rows, sem).wait()
                pltpu.sync_copy(rows, acc.at[didx], add=True)
                return 0
            lax.fori_loop(0, NCHUNK, body, 0)
            plsc.subcore_barrier()
            pltpu.sync_copy(acc.at[pl.ds(sid * RPW, RPW)],
                            o_hbm.at[pl.ds(sid * RPW, RPW)])

        @pl.when(cid == 0)
        def _():
            run(xlo_hbm, olo_hbm)

        @pl.when(cid == 1)
        def _():
            run(xhi_hbm, ohi_hbm)

    return agg_kernel(x_lo, x_hi, src, dst)


def _tc_prescale(feats, dsrc):
    BR = 2048

    def body(feats_ref, dsrc_ref, lo_ref, hi_ref):
        ns = lax.rsqrt(dsrc_ref[:, 0:1] + 1.0)
        xn = feats_ref[...] * ns
        lo_ref[...] = xn[:, :128]
        hi_ref[...] = xn[:, 128:]

    return pl.pallas_call(
        body,
        grid=(N_PAD // BR,),
        in_specs=[
            pl.BlockSpec((BR, D_IN), lambda r: (r, 0)),
            pl.BlockSpec((BR, DEGW), lambda r: (r, 0)),
        ],
        out_specs=[
            pl.BlockSpec((BR, 128), lambda r: (r, 0)),
            pl.BlockSpec((BR, 128), lambda r: (r, 0)),
        ],
        out_shape=(jax.ShapeDtypeStruct((N_PAD, 128), jnp.float32),
                   jax.ShapeDtypeStruct((N_PAD, 128), jnp.float32)),
    )(feats, dsrc)


def _tc_mid(alo, ahi, dsrc, ddst, W1, b1, W2):
    BR = 1024

    def body(alo_ref, ahi_ref, dsrc_ref, ddst_ref, W1_ref, b1_ref, W2_ref,
             glo_ref, ghi_ref):
        nd = lax.rsqrt(ddst_ref[:, 0:1] + 1.0)
        ns = lax.rsqrt(dsrc_ref[:, 0:1] + 1.0)
        a = jnp.concatenate([alo_ref[...], ahi_ref[...]], axis=1) * nd
        h = jnp.dot(a, W1_ref[...], preferred_element_type=jnp.float32)
        h = jnp.maximum(h + b1_ref[...], 0.0) * ns
        g = jnp.dot(h, W2_ref[...], preferred_element_type=jnp.float32)
        glo_ref[...] = g[:, :128]
        ghi_ref[...] = g[:, 128:]

    return pl.pallas_call(
        body,
        grid=(N_PAD // BR,),
        in_specs=[
            pl.BlockSpec((BR, 128), lambda r: (r, 0)),
            pl.BlockSpec((BR, 128), lambda r: (r, 0)),
            pl.BlockSpec((BR, DEGW), lambda r: (r, 0)),
            pl.BlockSpec((BR, DEGW), lambda r: (r, 0)),
            pl.BlockSpec((D_IN, D_HID), lambda r: (0, 0)),
            pl.BlockSpec((1, D_HID), lambda r: (0, 0)),
            pl.BlockSpec((D_HID, D_OUT), lambda r: (0, 0)),
        ],
        out_specs=[
            pl.BlockSpec((BR, 128), lambda r: (r, 0)),
            pl.BlockSpec((BR, 128), lambda r: (r, 0)),
        ],
        out_shape=(jax.ShapeDtypeStruct((N_PAD, 128), jnp.float32),
                   jax.ShapeDtypeStruct((N_PAD, 128), jnp.float32)),
    )(alo, ahi, dsrc, ddst, W1, b1, W2)


def _tc_final(blo, bhi, ddst, b2):
    BR = 2048

    def body(blo_ref, bhi_ref, ddst_ref, b2_ref, out_ref):
        nd = lax.rsqrt(ddst_ref[:, 0:1] + 1.0)
        agg = jnp.concatenate([blo_ref[...], bhi_ref[...]], axis=1)
        out_ref[...] = jnp.maximum(agg * nd + b2_ref[...], 0.0)

    return pl.pallas_call(
        body,
        grid=(N_PAD // BR,),
        in_specs=[
            pl.BlockSpec((BR, 128), lambda r: (r, 0)),
            pl.BlockSpec((BR, 128), lambda r: (r, 0)),
            pl.BlockSpec((BR, DEGW), lambda r: (r, 0)),
            pl.BlockSpec((1, D_OUT), lambda r: (0, 0)),
        ],
        out_specs=pl.BlockSpec((BR, D_OUT), lambda r: (r, 0)),
        out_shape=jax.ShapeDtypeStruct((N_PAD, D_OUT), jnp.float32),
    )(blo, bhi, ddst, b2)


def kernel(feats, edge_index, W1, b1, W2, b2):
    src = edge_index[0]
    dst = edge_index[1]
    feats_p = jnp.pad(feats, ((0, N_PAD - N), (0, 0)))
    zeros_rows = jnp.zeros((N_PAD, DEGW), jnp.float32)
    ones_rows = jnp.ones((CH, DEGW), jnp.float32)
    b1r = b1.reshape(1, D_HID)
    b2r = b2.reshape(1, D_OUT)

    dsrc, ddst = _sc_degrees(src, dst, zeros_rows, ones_rows)
    xlo, xhi = _tc_prescale(feats_p, dsrc)
    alo, ahi = _sc_aggregate(xlo, xhi, src, dst)
    glo, ghi = _tc_mid(alo, ahi, dsrc, ddst, W1, b1r, W2)
    blo, bhi = _sc_aggregate(glo, ghi, src, dst)
    return _tc_final(blo, bhi, ddst, b2r)[:N]
